# initial kernel scaffold (unmeasured)
import jax
import jax.numpy as jnp
from jax import lax
from jax.experimental import pallas as pl
from jax.experimental.pallas import tpu as pltpu

N_DEV = 8
SQ = 1024
D = 1024
H = 8
DH = 128
SKV = 1024
BLK = 64
SCALE = 0.08838834764831843


def _body(x_ref, wq_ref, kh_ref, vh_ref, wo_ref, out_ref,
          xbuf, accbuf, sx, rx, sa, ra, credit):
    my = lax.axis_index("i")
    left = lax.rem(my + N_DEV - 1, N_DEV)
    right = lax.rem(my + 1, N_DEV)

    barrier = pltpu.get_barrier_semaphore()
    for nbr in (left, right):
        pl.semaphore_signal(barrier, inc=1, device_id=(nbr,),
                            device_id_type=pl.DeviceIdType.MESH)
    pl.semaphore_wait(barrier, 2)

    rowb = lax.broadcasted_iota(jnp.int32, (SQ, SKV), 0) // BLK
    kb = lax.broadcasted_iota(jnp.int32, (SQ, SKV), 1) // BLK

    def partial(xj, j):
        q = jnp.dot(xj, wq_ref[...], preferred_element_type=jnp.float32)
        qb = rowb + j * (SQ // BLK)
        mask = (qb == kb) | (kb == 0) | (lax.rem(qb + kb, 3) == 0)
        ctxs = []
        for h in range(H):
            qh = q[:, h * DH:(h + 1) * DH]
            s = lax.dot_general(qh, kh_ref[h],
                                (((1,), (1,)), ((), ())),
                                preferred_element_type=jnp.float32) * SCALE
            s = jnp.where(mask, s, -1e9)
            m = jnp.max(s, axis=1, keepdims=True)
            w = jnp.exp(s - m)
            w = w / jnp.sum(w, axis=1, keepdims=True)
            ctxs.append(jnp.dot(w, vh_ref[h],
                                preferred_element_type=jnp.float32))
        ctx = jnp.concatenate(ctxs, axis=1)
        return jnp.dot(ctx, wo_ref[...], preferred_element_type=jnp.float32)

    xbuf[0] = x_ref[...]
    accbuf[0] = partial(x_ref[...], my)

    def hop(h, src_slot, dst_slot, send_x):
        rdmas = []
        if send_x:
            rdmas.append(pltpu.make_async_remote_copy(
                src_ref=xbuf.at[src_slot], dst_ref=xbuf.at[dst_slot],
                send_sem=sx.at[h], recv_sem=rx.at[h],
                device_id=(right,), device_id_type=pl.DeviceIdType.MESH))
        rdmas.append(pltpu.make_async_remote_copy(
            src_ref=accbuf.at[src_slot], dst_ref=accbuf.at[dst_slot],
            send_sem=sa.at[h], recv_sem=ra.at[h],
            device_id=(right,), device_id_type=pl.DeviceIdType.MESH))
        for r in rdmas:
            r.start()
        return rdmas

    for h in range(N_DEV - 1):
        s_, r_ = h % 2, (h + 1) % 2
        if h >= 1:
            pl.semaphore_wait(credit, 1)
        rdmas = hop(h, s_, r_, send_x=True)
        for r in rdmas:
            r.wait()
        pl.semaphore_signal(credit, inc=1, device_id=(left,),
                            device_id_type=pl.DeviceIdType.MESH)
        j = lax.rem(my + N_DEV - 1 - h, N_DEV)
        accbuf[r_] = accbuf[r_] + partial(xbuf[r_], j)

    pl.semaphore_wait(credit, 1)
    rdmas = hop(N_DEV - 1, 1, 0, send_x=False)
    for r in rdmas:
        r.wait()
    out_ref[...] = accbuf[0]


def kernel(x, Wq, K_ext, V_ext, Wo):
    my = lax.axis_index("i")
    Kh = jnp.transpose(
        lax.dynamic_slice_in_dim(K_ext[0], my * H, H, axis=1), (1, 0, 2))
    Vh = jnp.transpose(
        lax.dynamic_slice_in_dim(V_ext[0], my * H, H, axis=1), (1, 0, 2))

    out = pl.pallas_call(
        _body,
        out_shape=jax.ShapeDtypeStruct((SQ, D), jnp.float32),
        in_specs=[pl.BlockSpec(memory_space=pltpu.VMEM)] * 5,
        out_specs=pl.BlockSpec(memory_space=pltpu.VMEM),
        scratch_shapes=[
            pltpu.VMEM((2, SQ, D), jnp.float32),
            pltpu.VMEM((2, SQ, D), jnp.float32),
            pltpu.SemaphoreType.DMA((N_DEV,)),
            pltpu.SemaphoreType.DMA((N_DEV,)),
            pltpu.SemaphoreType.DMA((N_DEV,)),
            pltpu.SemaphoreType.DMA((N_DEV,)),
            pltpu.SemaphoreType.REGULAR,
        ],
        compiler_params=pltpu.CompilerParams(collective_id=0),
    )(x[0], Wq, Kh, Vh, Wo)
    return out[None]


# baseline (device time: 945179 ns/iter reference)
import jax
import jax.numpy as jnp
from jax import lax
from jax.experimental import pallas as pl
from jax.experimental.pallas import tpu as pltpu

N_DEV = 8
SQ = 1024
D = 1024
H = 8
DH = 128
SKV = 1024
BLK = 64
SCALE = 0.08838834764831843


def _body(x_ref, wq_ref, kh_ref, vh_ref, wo_ref, out_ref,
          xbuf, accbuf, qscr, ctxscr, sx, rx, sa, ra, credit):
    my = lax.axis_index("i")
    left = lax.rem(my + N_DEV - 1, N_DEV)
    right = lax.rem(my + 1, N_DEV)

    barrier = pltpu.get_barrier_semaphore()
    for nbr in (left, right):
        pl.semaphore_signal(barrier, inc=1, device_id=(nbr,),
                            device_id_type=pl.DeviceIdType.MESH)
    pl.semaphore_wait(barrier, 2)

    rowb = lax.broadcasted_iota(jnp.int32, (SQ, SKV), 0) // BLK
    kb = lax.broadcasted_iota(jnp.int32, (SQ, SKV), 1) // BLK

    def partial(xj, j):
        qscr[...] = jnp.dot(xj, wq_ref[...], preferred_element_type=jnp.float32)
        qb = rowb + j * (SQ // BLK)
        maskf = jnp.where(
            (qb == kb) | (kb == 0) | (lax.rem(qb + kb, 3) == 0), 0.0, -1e9
        ).astype(jnp.float32)

        def head(h, carry):
            qh = qscr[:, pl.ds(h * DH, DH)]
            s = lax.dot_general(qh, kh_ref[h],
                                (((1,), (1,)), ((), ())),
                                preferred_element_type=jnp.float32) * SCALE
            s = s + maskf
            m = jnp.max(s, axis=1, keepdims=True)
            w = jnp.exp(s - m)
            w = w / jnp.sum(w, axis=1, keepdims=True)
            ctxscr[:, pl.ds(h * DH, DH)] = jnp.dot(
                w, vh_ref[h], preferred_element_type=jnp.float32)
            return carry

        lax.fori_loop(0, H, head, 0)
        return jnp.dot(ctxscr[...], wo_ref[...],
                       preferred_element_type=jnp.float32)

    xbuf[0] = x_ref[...]
    accbuf[0] = partial(x_ref[...], my)

    def hop(h, src_slot, dst_slot, send_x):
        rdmas = []
        if send_x:
            rdmas.append(pltpu.make_async_remote_copy(
                src_ref=xbuf.at[src_slot], dst_ref=xbuf.at[dst_slot],
                send_sem=sx.at[h], recv_sem=rx.at[h],
                device_id=(right,), device_id_type=pl.DeviceIdType.MESH))
        rdmas.append(pltpu.make_async_remote_copy(
            src_ref=accbuf.at[src_slot], dst_ref=accbuf.at[dst_slot],
            send_sem=sa.at[h], recv_sem=ra.at[h],
            device_id=(right,), device_id_type=pl.DeviceIdType.MESH))
        for r in rdmas:
            r.start()
        return rdmas

    for h in range(N_DEV - 1):
        s_, r_ = h % 2, (h + 1) % 2
        if h >= 1:
            pl.semaphore_wait(credit, 1)
        rdmas = hop(h, s_, r_, send_x=True)
        for r in rdmas:
            r.wait()
        pl.semaphore_signal(credit, inc=1, device_id=(left,),
                            device_id_type=pl.DeviceIdType.MESH)
        j = lax.rem(my + N_DEV - 1 - h, N_DEV)
        accbuf[r_] = accbuf[r_] + partial(xbuf[r_], j)

    pl.semaphore_wait(credit, 1)
    rdmas = hop(N_DEV - 1, 1, 0, send_x=False)
    for r in rdmas:
        r.wait()
    out_ref[...] = accbuf[0]


def kernel(x, Wq, K_ext, V_ext, Wo):
    my = lax.axis_index("i")
    Kh = jnp.transpose(
        lax.dynamic_slice_in_dim(K_ext[0], my * H, H, axis=1), (1, 0, 2))
    Vh = jnp.transpose(
        lax.dynamic_slice_in_dim(V_ext[0], my * H, H, axis=1), (1, 0, 2))

    out = pl.pallas_call(
        _body,
        out_shape=jax.ShapeDtypeStruct((SQ, D), jnp.float32),
        in_specs=[pl.BlockSpec(memory_space=pltpu.VMEM)] * 5,
        out_specs=pl.BlockSpec(memory_space=pltpu.VMEM),
        scratch_shapes=[
            pltpu.VMEM((2, SQ, D), jnp.float32),
            pltpu.VMEM((2, SQ, D), jnp.float32),
            pltpu.VMEM((SQ, D), jnp.float32),
            pltpu.VMEM((SQ, D), jnp.float32),
            pltpu.SemaphoreType.DMA((N_DEV,)),
            pltpu.SemaphoreType.DMA((N_DEV,)),
            pltpu.SemaphoreType.DMA((N_DEV,)),
            pltpu.SemaphoreType.DMA((N_DEV,)),
            pltpu.SemaphoreType.REGULAR,
        ],
        compiler_params=pltpu.CompilerParams(
            collective_id=0, vmem_limit_bytes=100 * 1024 * 1024),
    )(x[0], Wq, Kh, Vh, Wo)
    return out[None]


# device time: 342917 ns/iter; 2.7563x vs baseline; 2.7563x over previous
import jax
import jax.numpy as jnp
from jax import lax
from jax.experimental import pallas as pl
from jax.experimental.pallas import tpu as pltpu

N_DEV = 8
SQ = 1024
HALF = SQ // 2
D = 1024
H = 8
DH = 128
SKV = 1024
BLK = 64
QB_PER_SHARD = SQ // BLK
SCALE = 0.08838834764831843
MESH = pl.DeviceIdType.MESH


def _body(x_ref, wq_ref, kh_ref, vh_ref, wo_ref, out_ref,
          xbufT, xbufB, accT, accB, qscr, ctxscr, dscr,
          sxR, rxR, sxL, rxL, saR, raR, saL, raL, creditR, creditL):
    my = lax.axis_index("i")
    left = lax.rem(my + N_DEV - 1, N_DEV)
    right = lax.rem(my + 1, N_DEV)

    barrier = pltpu.get_barrier_semaphore()
    for nbr in (left, right):
        pl.semaphore_signal(barrier, inc=1, device_id=(nbr,),
                            device_id_type=MESH)
    pl.semaphore_wait(barrier, 2)

    rowb = lax.broadcasted_iota(jnp.int32, (SQ, SKV), 0) // BLK
    kb = lax.broadcasted_iota(jnp.int32, (SQ, SKV), 1) // BLK
    is_top = lax.broadcasted_iota(jnp.int32, (SQ, SKV), 0) < HALF

    def x_rdmas(k):
        r = pltpu.make_async_remote_copy(
            src_ref=xbufT.at[k], dst_ref=xbufT.at[k + 1],
            send_sem=sxR.at[k], recv_sem=rxR.at[k],
            device_id=(right,), device_id_type=MESH)
        l = pltpu.make_async_remote_copy(
            src_ref=xbufB.at[k], dst_ref=xbufB.at[k + 1],
            send_sem=sxL.at[k], recv_sem=rxL.at[k],
            device_id=(left,), device_id_type=MESH)
        return r, l

    def acc_rdmas(h, s_, r_):
        r = pltpu.make_async_remote_copy(
            src_ref=accT.at[s_], dst_ref=accT.at[r_],
            send_sem=saR.at[h], recv_sem=raR.at[h],
            device_id=(right,), device_id_type=MESH)
        l = pltpu.make_async_remote_copy(
            src_ref=accB.at[s_], dst_ref=accB.at[r_],
            send_sem=saL.at[h], recv_sem=raL.at[h],
            device_id=(left,), device_id_type=MESH)
        return r, l

    def partial(k, jR, jL):
        qscr[:HALF, :] = jnp.dot(
            xbufT[k], wq_ref[...],
            preferred_element_type=jnp.float32).astype(jnp.bfloat16)
        qscr[HALF:, :] = jnp.dot(
            xbufB[k], wq_ref[...],
            preferred_element_type=jnp.float32).astype(jnp.bfloat16)
        qb = rowb + jnp.where(is_top, jR * QB_PER_SHARD, jL * QB_PER_SHARD)
        maskf = jnp.where(
            (qb == kb) | (kb == 0) | (lax.rem(qb + kb, 3) == 0), 0.0, -1e9
        ).astype(jnp.float32)

        def head(h, carry):
            qh = qscr[:, pl.ds(h * DH, DH)]
            s = lax.dot_general(qh, kh_ref[h],
                                (((1,), (1,)), ((), ())),
                                preferred_element_type=jnp.float32) * SCALE
            s = s + maskf
            m = jnp.max(s, axis=1, keepdims=True)
            w = jnp.exp(s - m)
            w = (w / jnp.sum(w, axis=1, keepdims=True)).astype(jnp.bfloat16)
            ctxscr[:, pl.ds(h * DH, DH)] = jnp.dot(
                w, vh_ref[h], preferred_element_type=jnp.float32
            ).astype(jnp.bfloat16)
            return carry

        lax.fori_loop(0, H, head, 0)
        dscr[...] = jnp.dot(ctxscr[...], wo_ref[...],
                            preferred_element_type=jnp.float32)

    xbufT[0] = x_ref[:HALF, :]
    xbufB[0] = x_ref[HALF:, :]
    xr0, xl0 = x_rdmas(0)
    xr0.start()
    xl0.start()
    partial(0, my, my)
    accT[0] = dscr[:HALF, :]
    accB[0] = dscr[HALF:, :]

    for h in range(N_DEV - 1):
        s_, r_ = h % 2, (h + 1) % 2
        xr, xl = x_rdmas(h)
        xr.wait_recv()
        xl.wait_recv()
        if h + 1 <= N_DEV - 2:
            xr2, xl2 = x_rdmas(h + 1)
            xr2.start()
            xl2.start()
        if h >= 1:
            pl.semaphore_wait(creditR, 1)
            pl.semaphore_wait(creditL, 1)
        ar, al = acc_rdmas(h, s_, r_)
        ar.start()
        al.start()
        jR = lax.rem(my + N_DEV - (h + 1), N_DEV)
        jL = lax.rem(my + h + 1, N_DEV)
        partial(h + 1, jR, jL)
        ar.wait()
        al.wait()
        pl.semaphore_signal(creditR, inc=1, device_id=(left,),
                            device_id_type=MESH)
        pl.semaphore_signal(creditL, inc=1, device_id=(right,),
                            device_id_type=MESH)
        accT[r_] = accT[r_] + dscr[:HALF, :]
        accB[r_] = accB[r_] + dscr[HALF:, :]

    pl.semaphore_wait(creditR, 1)
    pl.semaphore_wait(creditL, 1)
    ar, al = acc_rdmas(N_DEV - 1, 1, 0)
    ar.start()
    al.start()
    ar.wait()
    al.wait()
    for k in range(N_DEV - 1):
        xr, xl = x_rdmas(k)
        xr.wait_send()
        xl.wait_send()
    out_ref[:HALF, :] = accT[0]
    out_ref[HALF:, :] = accB[0]


def kernel(x, Wq, K_ext, V_ext, Wo):
    my = lax.axis_index("i")
    Kh = jnp.transpose(
        lax.dynamic_slice_in_dim(K_ext[0], my * H, H, axis=1),
        (1, 0, 2)).astype(jnp.bfloat16)
    Vh = jnp.transpose(
        lax.dynamic_slice_in_dim(V_ext[0], my * H, H, axis=1),
        (1, 0, 2)).astype(jnp.bfloat16)

    out = pl.pallas_call(
        _body,
        out_shape=jax.ShapeDtypeStruct((SQ, D), jnp.float32),
        in_specs=[pl.BlockSpec(memory_space=pltpu.VMEM)] * 5,
        out_specs=pl.BlockSpec(memory_space=pltpu.VMEM),
        scratch_shapes=[
            pltpu.VMEM((N_DEV, HALF, D), jnp.bfloat16),
            pltpu.VMEM((N_DEV, HALF, D), jnp.bfloat16),
            pltpu.VMEM((2, HALF, D), jnp.float32),
            pltpu.VMEM((2, HALF, D), jnp.float32),
            pltpu.VMEM((SQ, D), jnp.bfloat16),
            pltpu.VMEM((SQ, D), jnp.bfloat16),
            pltpu.VMEM((SQ, D), jnp.float32),
            pltpu.SemaphoreType.DMA((N_DEV,)),
            pltpu.SemaphoreType.DMA((N_DEV,)),
            pltpu.SemaphoreType.DMA((N_DEV,)),
            pltpu.SemaphoreType.DMA((N_DEV,)),
            pltpu.SemaphoreType.DMA((N_DEV,)),
            pltpu.SemaphoreType.DMA((N_DEV,)),
            pltpu.SemaphoreType.DMA((N_DEV,)),
            pltpu.SemaphoreType.DMA((N_DEV,)),
            pltpu.SemaphoreType.REGULAR,
            pltpu.SemaphoreType.REGULAR,
        ],
        compiler_params=pltpu.CompilerParams(
            collective_id=0, vmem_limit_bytes=100 * 1024 * 1024),
    )(x[0].astype(jnp.bfloat16), Wq.astype(jnp.bfloat16), Kh, Vh,
      Wo.astype(jnp.bfloat16))
    return out[None]


# device time: 272429 ns/iter; 3.4695x vs baseline; 1.2587x over previous
import jax
import jax.numpy as jnp
from jax import lax
from jax.experimental import pallas as pl
from jax.experimental.pallas import tpu as pltpu

N_DEV = 8
SQ = 1024
HALF = SQ // 2
D = 1024
H = 8
DH = 128
SKV = 1024
BLK = 64
QB_PER_SHARD = SQ // BLK
SCALE = 0.08838834764831843
MESH = pl.DeviceIdType.MESH


def _body(x_ref, wq_ref, kh_ref, vh_ref, wo_ref, out_ref,
          xbufT, xbufB, accT, accB, qscr, ctxscr, dscr,
          sxR, rxR, sxL, rxL, saR, raR, saL, raL, creditR, creditL):
    my = lax.axis_index("i")
    left = lax.rem(my + N_DEV - 1, N_DEV)
    right = lax.rem(my + 1, N_DEV)

    barrier = pltpu.get_barrier_semaphore()
    for nbr in (left, right):
        pl.semaphore_signal(barrier, inc=1, device_id=(nbr,),
                            device_id_type=MESH)
    pl.semaphore_wait(barrier, 2)

    rowb = lax.broadcasted_iota(jnp.int32, (SQ, SKV), 0) // BLK
    kb = lax.broadcasted_iota(jnp.int32, (SQ, SKV), 1) // BLK
    is_top = lax.broadcasted_iota(jnp.int32, (SQ, SKV), 0) < HALF

    def x_rdmas(k):
        r = pltpu.make_async_remote_copy(
            src_ref=xbufT.at[k], dst_ref=xbufT.at[k + 1],
            send_sem=sxR.at[k], recv_sem=rxR.at[k],
            device_id=(right,), device_id_type=MESH)
        l = pltpu.make_async_remote_copy(
            src_ref=xbufB.at[k], dst_ref=xbufB.at[k + 1],
            send_sem=sxL.at[k], recv_sem=rxL.at[k],
            device_id=(left,), device_id_type=MESH)
        return r, l

    def acc_rdmas(h, s_, r_):
        r = pltpu.make_async_remote_copy(
            src_ref=accT.at[s_], dst_ref=accT.at[r_],
            send_sem=saR.at[h], recv_sem=raR.at[h],
            device_id=(right,), device_id_type=MESH)
        l = pltpu.make_async_remote_copy(
            src_ref=accB.at[s_], dst_ref=accB.at[r_],
            send_sem=saL.at[h], recv_sem=raL.at[h],
            device_id=(left,), device_id_type=MESH)
        return r, l

    def partial(k, jR, jL):
        qscr[:HALF, :] = jnp.dot(
            xbufT[k], wq_ref[...],
            preferred_element_type=jnp.float32).astype(jnp.bfloat16)
        qscr[HALF:, :] = jnp.dot(
            xbufB[k], wq_ref[...],
            preferred_element_type=jnp.float32).astype(jnp.bfloat16)
        qb = rowb + jnp.where(is_top, jR * QB_PER_SHARD, jL * QB_PER_SHARD)
        maskf = jnp.where(
            (qb == kb) | (kb == 0) | (lax.rem(qb + kb, 3) == 0), 0.0, -1e9
        ).astype(jnp.float32)

        def head(h, carry):
            qh = qscr[:, pl.ds(h * DH, DH)]
            s = lax.dot_general(qh, kh_ref[h],
                                (((1,), (1,)), ((), ())),
                                preferred_element_type=jnp.float32) * SCALE
            w = jnp.exp(s + maskf)
            w = (w / jnp.sum(w, axis=1, keepdims=True)).astype(jnp.bfloat16)
            ctxscr[:, pl.ds(h * DH, DH)] = jnp.dot(
                w, vh_ref[h], preferred_element_type=jnp.float32
            ).astype(jnp.bfloat16)
            return carry

        lax.fori_loop(0, H, head, 0)
        dscr[...] = jnp.dot(ctxscr[...], wo_ref[...],
                            preferred_element_type=jnp.float32)

    xbufT[0] = x_ref[:HALF, :]
    xbufB[0] = x_ref[HALF:, :]
    xr0, xl0 = x_rdmas(0)
    xr0.start()
    xl0.start()
    partial(0, my, my)
    accT[0] = dscr[:HALF, :].astype(jnp.bfloat16)
    accB[0] = dscr[HALF:, :].astype(jnp.bfloat16)

    for h in range(N_DEV - 1):
        s_, r_ = h % 2, (h + 1) % 2
        xr, xl = x_rdmas(h)
        xr.wait_recv()
        xl.wait_recv()
        if h + 1 <= N_DEV - 2:
            xr2, xl2 = x_rdmas(h + 1)
            xr2.start()
            xl2.start()
        if h >= 1:
            pl.semaphore_wait(creditR, 1)
            pl.semaphore_wait(creditL, 1)
        ar, al = acc_rdmas(h, s_, r_)
        ar.start()
        al.start()
        jR = lax.rem(my + N_DEV - (h + 1), N_DEV)
        jL = lax.rem(my + h + 1, N_DEV)
        partial(h + 1, jR, jL)
        ar.wait()
        al.wait()
        pl.semaphore_signal(creditR, inc=1, device_id=(left,),
                            device_id_type=MESH)
        pl.semaphore_signal(creditL, inc=1, device_id=(right,),
                            device_id_type=MESH)
        accT[r_] = (accT[r_].astype(jnp.float32)
                    + dscr[:HALF, :]).astype(jnp.bfloat16)
        accB[r_] = (accB[r_].astype(jnp.float32)
                    + dscr[HALF:, :]).astype(jnp.bfloat16)

    pl.semaphore_wait(creditR, 1)
    pl.semaphore_wait(creditL, 1)
    ar, al = acc_rdmas(N_DEV - 1, 1, 0)
    ar.start()
    al.start()
    ar.wait()
    al.wait()
    for k in range(N_DEV - 1):
        xr, xl = x_rdmas(k)
        xr.wait_send()
        xl.wait_send()
    out_ref[:HALF, :] = accT[0].astype(jnp.float32)
    out_ref[HALF:, :] = accB[0].astype(jnp.float32)


def kernel(x, Wq, K_ext, V_ext, Wo):
    my = lax.axis_index("i")
    Kh = jnp.transpose(
        lax.dynamic_slice_in_dim(K_ext[0], my * H, H, axis=1),
        (1, 0, 2)).astype(jnp.bfloat16)
    Vh = jnp.transpose(
        lax.dynamic_slice_in_dim(V_ext[0], my * H, H, axis=1),
        (1, 0, 2)).astype(jnp.bfloat16)

    out = pl.pallas_call(
        _body,
        out_shape=jax.ShapeDtypeStruct((SQ, D), jnp.float32),
        in_specs=[pl.BlockSpec(memory_space=pltpu.VMEM)] * 5,
        out_specs=pl.BlockSpec(memory_space=pltpu.VMEM),
        scratch_shapes=[
            pltpu.VMEM((N_DEV, HALF, D), jnp.bfloat16),
            pltpu.VMEM((N_DEV, HALF, D), jnp.bfloat16),
            pltpu.VMEM((2, HALF, D), jnp.bfloat16),
            pltpu.VMEM((2, HALF, D), jnp.bfloat16),
            pltpu.VMEM((SQ, D), jnp.bfloat16),
            pltpu.VMEM((SQ, D), jnp.bfloat16),
            pltpu.VMEM((SQ, D), jnp.float32),
            pltpu.SemaphoreType.DMA((N_DEV,)),
            pltpu.SemaphoreType.DMA((N_DEV,)),
            pltpu.SemaphoreType.DMA((N_DEV,)),
            pltpu.SemaphoreType.DMA((N_DEV,)),
            pltpu.SemaphoreType.DMA((N_DEV,)),
            pltpu.SemaphoreType.DMA((N_DEV,)),
            pltpu.SemaphoreType.DMA((N_DEV,)),
            pltpu.SemaphoreType.DMA((N_DEV,)),
            pltpu.SemaphoreType.REGULAR,
            pltpu.SemaphoreType.REGULAR,
        ],
        compiler_params=pltpu.CompilerParams(
            collective_id=0, vmem_limit_bytes=100 * 1024 * 1024),
    )(x[0].astype(jnp.bfloat16), Wq.astype(jnp.bfloat16), Kh, Vh,
      Wo.astype(jnp.bfloat16))
    return out[None]
